# Initial kernel scaffold; baseline (speedup 1.0000x reference)
import jax
import jax.numpy as jnp
from jax.experimental import pallas as pl


def kernel(input_features, bb1_table, reaction_table, W1, b1, g1, be1, rm1, rv1, W2, b2, g2, be2, rm2, rv2):
    raise NotImplementedError("write your pallas kernel here")


# speedup vs baseline: 1.0118x; 1.0118x over previous
import functools

import jax
import jax.numpy as jnp
from jax import lax
from jax.experimental import pallas as pl
from jax.experimental.pallas import tpu as pltpu
from jax.experimental.pallas import tpu_sc as plsc

_B = 16384
_V = 1000000
_D = 64
_H = 128

_NC = 2
_NS = 16
_NW = _NC * _NS
_BPW = _B // _NW
_L = 16
_CH = 256
_NCHUNK = _BPW // _CH


@functools.cache
def _make_sc_gather2():
    mesh = plsc.VectorSubcoreMesh(core_axis_name="c", subcore_axis_name="s")

    @functools.partial(
        pl.kernel,
        mesh=mesh,
        out_type=jax.ShapeDtypeStruct((_B, _D), jnp.float32),
        scratch_types=[
            pltpu.VMEM((_BPW,), jnp.int32),
            pltpu.VMEM((_BPW,), jnp.int32),
            pltpu.VMEM((_CH, _D), jnp.float32),
            pltpu.VMEM((_CH, _D), jnp.float32),
            pltpu.SemaphoreType.DMA,
        ],
        compiler_params=pltpu.CompilerParams(needs_layout_passes=False),
    )
    def _sc_gather2(t1_hbm, t2_hbm, i0_hbm, i1_hbm, o_hbm,
                    raw0_v, raw1_v, r1_v, r2_v, sem):
        wid = lax.axis_index("s") * _NC + lax.axis_index("c")
        base = wid * _BPW
        pltpu.sync_copy(i0_hbm.at[wid], raw0_v)
        pltpu.sync_copy(i1_hbm.at[wid], raw1_v)

        for ch in range(_NCHUNK):
            ibase = ch * _CH

            def issue(g):
                g16 = g * _L
                v0 = raw0_v[pl.ds(ibase + g16, _L)]
                t0 = v0 >> 3
                s0 = v0 & 7
                v1 = raw1_v[pl.ds(ibase + g16, _L)]
                t1 = v1 >> 3
                s1 = v1 & 7
                for l in range(_L):
                    pltpu.async_copy(t1_hbm.at[t0[l], s0[l]],
                                     r1_v.at[g16 + l], sem)
                    pltpu.async_copy(t2_hbm.at[t1[l], s1[l]],
                                     r2_v.at[g16 + l], sem)

            pl.loop(0, _CH // _L)(issue)

            def drain(i):
                pltpu.make_async_copy(t1_hbm.at[0, 0], r1_v.at[i], sem).wait()
                pltpu.make_async_copy(t2_hbm.at[0, 0], r2_v.at[i], sem).wait()

            pl.loop(0, _CH)(drain)

            def accum(i):
                for j in range(_D // _L):
                    sl = pl.ds(j * _L, _L)
                    r1_v[i, sl] = r1_v[i, sl] + r2_v[i, sl]

            pl.loop(0, _CH)(accum)
            pltpu.sync_copy(r1_v, o_hbm.at[pl.ds(base + ibase, _CH)])

    return _sc_gather2


def _tc_mlp_body(e_ref, W1_ref, b1_ref, s1_ref, t1_ref,
                 W2_ref, b2_ref, s2_ref, t2_ref, o_ref):
    h = jnp.dot(e_ref[...], W1_ref[...], preferred_element_type=jnp.float32)
    h = h + b1_ref[...]
    h = jnp.where(h > 0, h, 0.01 * h)
    h = h * s1_ref[...] + t1_ref[...]
    h = jnp.dot(h, W2_ref[...], preferred_element_type=jnp.float32) + b2_ref[...]
    h = jnp.where(h > 0, h, 0.01 * h)
    o_ref[...] = h * s2_ref[...] + t2_ref[...]


_BM = 2048


def _tc_mlp(e, W1, b1, s1, t1, W2, b2, s2, t2):
    grid = (_B // _BM,)
    full = lambda shape: pl.BlockSpec(shape, lambda i: (0, 0))
    return pl.pallas_call(
        _tc_mlp_body,
        grid=grid,
        in_specs=[
            pl.BlockSpec((_BM, _D), lambda i: (i, 0)),
            full((_D, 2 * _D)),
            full((1, 2 * _D)),
            full((1, 2 * _D)),
            full((1, 2 * _D)),
            full((2 * _D, _H)),
            full((1, _H)),
            full((1, _H)),
            full((1, _H)),
        ],
        out_specs=pl.BlockSpec((_BM, _H), lambda i: (i, 0)),
        out_shape=jax.ShapeDtypeStruct((_B, _H), jnp.float32),
    )(e, W1, b1, s1, t1, W2, b2, s2, t2)


def kernel(input_features, bb1_table, reaction_table, W1, b1, g1, be1, rm1,
           rv1, W2, b2, g2, be2, rm2, rv2):
    t1 = bb1_table.reshape(_V // 8, 8, _D)
    t2 = reaction_table.reshape(_V // 8, 8, _D)
    idx = input_features.reshape(_NW, _BPW, 2)
    idx0 = idx[..., 0]
    idx1 = idx[..., 1]
    e = _make_sc_gather2()(t1, t2, idx0, idx1)
    s1 = g1 * lax.rsqrt(rv1 + 1e-5)
    t1v = be1 - rm1 * s1
    s2 = g2 * lax.rsqrt(rv2 + 1e-5)
    t2v = be2 - rm2 * s2
    r = lambda v: v.reshape(1, -1)
    return _tc_mlp(e, W1, r(b1), r(s1), r(t1v), W2, r(b2), r(s2), r(t2v))
